# baseline (speedup 1.0000x reference)
import functools

import jax
import jax.numpy as jnp
from jax import lax
from jax.experimental import pallas as pl
from jax.experimental.pallas import tpu as pltpu
from jax.experimental.pallas import tpu_sc as plsc

N = 10000
E = 320000
G = 100
STEPS = 3
EPS = 0.01
CLAMP_STEP = 0.1

B = 256
NPAD = 10240
T = NPAD // B
NTILES = 32
EPT = E // NTILES
LANES = 16
UNROLL = 5
PAD_SENT = 2 ** 30


def _prep_body(batch_ref, sqrtcnt_ref, invcnt_ref):
    b = batch_ref[...]
    g = lax.broadcasted_iota(jnp.int32, (G, 1), 0)
    m = (b == g).astype(jnp.float32)
    cnt_g = jnp.sum(m, axis=1, keepdims=True)
    cnode = jnp.sum(m * cnt_g, axis=0, keepdims=True)
    c = jnp.maximum(cnode, 1.0)
    sqrtcnt_ref[...] = jnp.sqrt(c)
    invcnt_ref[...] = 1.0 / c


_prep = pl.pallas_call(
    _prep_body,
    out_shape=(
        jax.ShapeDtypeStruct((1, NPAD), jnp.float32),
        jax.ShapeDtypeStruct((1, NPAD), jnp.float32),
    ),
)


def _sc_attract_body(co_hbm, sq_hbm, row_hbm, col_hbm, out_hbm,
                     cov, sqv, rv, cv, fxv, fyv):
    wid = lax.axis_index("s") * 2 + lax.axis_index("c")
    base = wid * EPT
    pltpu.sync_copy(co_hbm, cov)
    pltpu.sync_copy(sq_hbm, sqv)
    pltpu.sync_copy(row_hbm.at[pl.ds(base, EPT)], rv)
    pltpu.sync_copy(col_hbm.at[pl.ds(base, EPT)], cv)

    zero16 = jnp.zeros((LANES,), jnp.float32)

    def _zero(i, carry):
        fxv[pl.ds(i * LANES, LANES)] = zero16
        fyv[pl.ds(i * LANES, LANES)] = zero16
        return carry

    lax.fori_loop(0, NPAD // LANES, _zero, 0)

    def _edges(i, carry):
        for u in range(UNROLL):
            o = (i * UNROLL + u) * LANES
            r = rv[pl.ds(o, LANES)]
            c = cv[pl.ds(o, LANES)]
            xr = plsc.load_gather(cov, [r])
            yr = plsc.load_gather(cov, [r + NPAD])
            xc = plsc.load_gather(cov, [c])
            yc = plsc.load_gather(cov, [c + NPAD])
            sq = plsc.load_gather(sqv, [r])
            dx = xr - xc
            dy = yr - yc
            d2 = dx * dx + dy * dy + 1e-20
            ib = plsc.bitcast(d2, jnp.int32)
            y = plsc.bitcast(jnp.int32(0x5F3759DF) - (ib >> 1), jnp.float32)
            h = 0.5 * d2
            y = y * (1.5 - h * y * y)
            y = y * (1.5 - h * y * y)
            coef = -(d2 * y + EPS) * sq
            ax = coef * dx
            ay = coef * dy
            plsc.addupdate_scatter(fxv, [r], ax)
            plsc.addupdate_scatter(fyv, [r], ay)
            plsc.addupdate_scatter(fxv, [c], -ax)
            plsc.addupdate_scatter(fyv, [c], -ay)
        return carry

    lax.fori_loop(0, EPT // (LANES * UNROLL), _edges, 0)

    pltpu.sync_copy(fxv, out_hbm.at[0, wid])
    pltpu.sync_copy(fyv, out_hbm.at[1, wid])


@functools.cache
def _sc_attract_kernel():
    mesh = plsc.VectorSubcoreMesh(core_axis_name="c", subcore_axis_name="s")
    return pl.kernel(
        _sc_attract_body,
        mesh=mesh,
        compiler_params=pltpu.CompilerParams(needs_layout_passes=False),
        out_type=jax.ShapeDtypeStruct((2, NTILES, NPAD), jnp.float32),
        scratch_types=[
            pltpu.VMEM((2 * NPAD,), jnp.float32),
            pltpu.VMEM((NPAD,), jnp.float32),
            pltpu.VMEM((EPT,), jnp.int32),
            pltpu.VMEM((EPT,), jnp.int32),
            pltpu.VMEM((NPAD,), jnp.float32),
            pltpu.VMEM((NPAD,), jnp.float32),
        ],
    )


def _rep_body(co_ref, bf_ref, ic_ref, coT_ref, bfT_ref, lo_ref, hi_ref,
              rep_ref):
    i = pl.program_id(0)
    xi = co_ref[0:1, :]
    yi = co_ref[1:2, :]
    bi = bf_ref[...]
    ki2 = ic_ref[...]
    gi = B * i + lax.broadcasted_iota(jnp.int32, (1, B), 1)

    def jbody(j, carry):
        sx, sy = carry
        off = j * B
        xj = coT_ref[pl.ds(off, B), 0:1]
        yj = coT_ref[pl.ds(off, B), 1:2]
        bj = bfT_ref[pl.ds(off, B), :]
        gj = B * j + lax.broadcasted_iota(jnp.int32, (B, 1), 0)
        dx = xi - xj
        dy = yi - yj
        eye = (gi == gj).astype(jnp.float32)
        d2 = dx * dx + dy * dy + eye
        dist = jnp.sqrt(d2) + EPS
        w = jnp.where(bi == bj, 1.0 / (dist * dist), 0.0)
        sx = sx + jnp.sum(w * dx, axis=0, keepdims=True)
        sy = sy + jnp.sum(w * dy, axis=0, keepdims=True)
        return sx, sy

    z = jnp.zeros((1, B), jnp.float32)
    sx, sy = lax.fori_loop(lo_ref[i], hi_ref[i], jbody, (z, z))
    rep_ref[0:1, :] = ki2 * sx
    rep_ref[1:2, :] = ki2 * sy


_smem = pl.BlockSpec(memory_space=pltpu.SMEM)

_rep = pl.pallas_call(
    _rep_body,
    grid=(T,),
    in_specs=[
        pl.BlockSpec((2, B), lambda i: (0, i)),
        pl.BlockSpec((1, B), lambda i: (0, i)),
        pl.BlockSpec((1, B), lambda i: (0, i)),
        pl.BlockSpec((NPAD, 2), lambda i: (0, 0)),
        pl.BlockSpec((NPAD, 1), lambda i: (0, 0)),
        _smem,
        _smem,
    ],
    out_specs=pl.BlockSpec((2, B), lambda i: (0, i)),
    out_shape=jax.ShapeDtypeStruct((2, NPAD), jnp.float32),
)


def _upd_body(co_ref, rep_ref, par_ref, alpha_ref, out_ref):
    fax = jnp.sum(par_ref[0], axis=0, keepdims=True)
    fay = jnp.sum(par_ref[1], axis=0, keepdims=True)
    a = alpha_ref[0, 0]
    dxt = a * (fax + rep_ref[0:1, :])
    dyt = a * (fay + rep_ref[1:2, :])
    nrm = jnp.sqrt(dxt * dxt + dyt * dyt + 1e-20)
    scale = jnp.minimum(CLAMP_STEP / (nrm + 1e-9), 1.0)
    out_ref[0:1, :] = co_ref[0:1, :] + dxt * scale
    out_ref[1:2, :] = co_ref[1:2, :] + dyt * scale


_upd = pl.pallas_call(
    _upd_body,
    in_specs=[
        pl.BlockSpec((2, NPAD), lambda: (0, 0)),
        pl.BlockSpec((2, NPAD), lambda: (0, 0)),
        pl.BlockSpec((2, NTILES, NPAD), lambda: (0, 0, 0)),
        _smem,
    ],
    out_specs=pl.BlockSpec((2, NPAD), lambda: (0, 0)),
    out_shape=jax.ShapeDtypeStruct((2, NPAD), jnp.float32),
)


def kernel(x, alpha, edge_index, batch):
    row = edge_index[0]
    col = edge_index[1]
    coT = jnp.pad(x[:, -2:], ((0, NPAD - N), (0, 0)))
    co = coT.T
    batch_p = jnp.pad(batch, (0, NPAD - N), constant_values=PAD_SENT)
    bf = batch_p.astype(jnp.float32)

    sqrtcnt, invcnt = _prep(batch_p.reshape(1, NPAD))

    tiles = bf.reshape(T, B)
    tmin = tiles[:, 0]
    tmax = tiles[:, -1]
    lo = jnp.searchsorted(tmax, tmin, side="left").astype(jnp.int32)
    hi = jnp.searchsorted(tmin, tmax, side="right").astype(jnp.int32)

    alpha_s = jnp.reshape(alpha, (1, 1)).astype(jnp.float32)
    sq1 = sqrtcnt.reshape(NPAD)
    bf2 = bf.reshape(1, NPAD)
    bfT = bf.reshape(NPAD, 1)

    for step in range(STEPS):
        par = _sc_attract_kernel()(co.reshape(2 * NPAD), sq1, row, col)
        rep = _rep(co, bf2, invcnt, coT, bfT, lo, hi)
        co = _upd(co, rep, par, alpha_s)
        if step + 1 < STEPS:
            coT = co.T
    return co[:, :N].T


# speedup vs baseline: 1.0493x; 1.0493x over previous
import functools

import jax
import jax.numpy as jnp
from jax import lax
from jax.experimental import pallas as pl
from jax.experimental.pallas import tpu as pltpu
from jax.experimental.pallas import tpu_sc as plsc

N = 10000
E = 320000
G = 100
STEPS = 3
EPS = 0.01
CLAMP_STEP = 0.1

B = 256
NPAD = 10240
T = NPAD // B
NTILES = 32
EPT = E // NTILES
LANES = 16
UNROLL = 5
PAD_SENT = 2 ** 30


def _prep_body(batch_ref, sqrtcnt_ref, invcnt_ref):
    b = batch_ref[...]
    g = lax.broadcasted_iota(jnp.int32, (G, 1), 0)
    m = (b == g).astype(jnp.float32)
    cnt_g = jnp.sum(m, axis=1, keepdims=True)
    cnode = jnp.sum(m * cnt_g, axis=0, keepdims=True)
    c = jnp.maximum(cnode, 1.0)
    sqrtcnt_ref[...] = jnp.sqrt(c)
    invcnt_ref[...] = 1.0 / c


_prep = pl.pallas_call(
    _prep_body,
    out_shape=(
        jax.ShapeDtypeStruct((1, NPAD), jnp.float32),
        jax.ShapeDtypeStruct((1, NPAD), jnp.float32),
    ),
)


def _sc_attract_body(co_hbm, sq_hbm, ei_hbm, out_hbm,
                     cov, sqv, rv, cv, fxv, fyv):
    wid = lax.axis_index("s") * 2 + lax.axis_index("c")
    base = wid * EPT
    pltpu.sync_copy(co_hbm, cov)
    pltpu.sync_copy(sq_hbm, sqv)
    pltpu.sync_copy(ei_hbm.at[pl.ds(base, EPT)], rv)
    pltpu.sync_copy(ei_hbm.at[pl.ds(E + base, EPT)], cv)

    zero16 = jnp.zeros((LANES,), jnp.float32)

    def _zero(i, carry):
        fxv[pl.ds(i * LANES, LANES)] = zero16
        fyv[pl.ds(i * LANES, LANES)] = zero16
        return carry

    lax.fori_loop(0, NPAD // LANES, _zero, 0)

    def _edges(i, carry):
        for u in range(UNROLL):
            o = (i * UNROLL + u) * LANES
            r = rv[pl.ds(o, LANES)]
            c = cv[pl.ds(o, LANES)]
            xr = plsc.load_gather(cov, [r])
            yr = plsc.load_gather(cov, [r + NPAD])
            xc = plsc.load_gather(cov, [c])
            yc = plsc.load_gather(cov, [c + NPAD])
            sq = plsc.load_gather(sqv, [r])
            dx = xr - xc
            dy = yr - yc
            d2 = dx * dx + dy * dy + 1e-20
            ib = plsc.bitcast(d2, jnp.int32)
            y = plsc.bitcast(jnp.int32(0x5F3759DF) - (ib >> 1), jnp.float32)
            h = 0.5 * d2
            y = y * (1.5 - h * y * y)
            y = y * (1.5 - h * y * y)
            coef = -(d2 * y + EPS) * sq
            ax = coef * dx
            ay = coef * dy
            plsc.addupdate_scatter(fxv, [r], ax)
            plsc.addupdate_scatter(fyv, [r], ay)
            plsc.addupdate_scatter(fxv, [c], -ax)
            plsc.addupdate_scatter(fyv, [c], -ay)
        return carry

    lax.fori_loop(0, EPT // (LANES * UNROLL), _edges, 0)

    pltpu.sync_copy(fxv, out_hbm.at[0, wid])
    pltpu.sync_copy(fyv, out_hbm.at[1, wid])


@functools.cache
def _sc_attract_kernel():
    mesh = plsc.VectorSubcoreMesh(core_axis_name="c", subcore_axis_name="s")
    return pl.kernel(
        _sc_attract_body,
        mesh=mesh,
        compiler_params=pltpu.CompilerParams(needs_layout_passes=False),
        out_type=jax.ShapeDtypeStruct((2, NTILES, NPAD), jnp.float32),
        scratch_types=[
            pltpu.VMEM((2 * NPAD,), jnp.float32),
            pltpu.VMEM((NPAD,), jnp.float32),
            pltpu.VMEM((EPT,), jnp.int32),
            pltpu.VMEM((EPT,), jnp.int32),
            pltpu.VMEM((NPAD,), jnp.float32),
            pltpu.VMEM((NPAD,), jnp.float32),
        ],
    )


def _rep_body(co_ref, bf_ref, ic_ref, coT_ref, bfT_ref, lo_ref, hi_ref,
              rep_ref):
    i = pl.program_id(0)
    xi = co_ref[0:1, :]
    yi = co_ref[1:2, :]
    bi = bf_ref[...]
    ki2 = ic_ref[...]
    gi = B * i + lax.broadcasted_iota(jnp.int32, (1, B), 1)

    def jbody(j, carry):
        sx, sy = carry
        off = j * B
        xj = coT_ref[pl.ds(off, B), 0:1]
        yj = coT_ref[pl.ds(off, B), 1:2]
        bj = bfT_ref[pl.ds(off, B), :]
        gj = B * j + lax.broadcasted_iota(jnp.int32, (B, 1), 0)
        dx = xi - xj
        dy = yi - yj
        eye = (gi == gj).astype(jnp.float32)
        d2 = dx * dx + dy * dy + eye
        dist = jnp.sqrt(d2) + EPS
        w = jnp.where(bi == bj, 1.0 / (dist * dist), 0.0)
        sx = sx + jnp.sum(w * dx, axis=0, keepdims=True)
        sy = sy + jnp.sum(w * dy, axis=0, keepdims=True)
        return sx, sy

    z = jnp.zeros((1, B), jnp.float32)
    sx, sy = lax.fori_loop(lo_ref[i], hi_ref[i], jbody, (z, z))
    rep_ref[0:1, :] = ki2 * sx
    rep_ref[1:2, :] = ki2 * sy


_smem = pl.BlockSpec(memory_space=pltpu.SMEM)

_rep = pl.pallas_call(
    _rep_body,
    grid=(T,),
    in_specs=[
        pl.BlockSpec((2, B), lambda i: (0, i)),
        pl.BlockSpec((1, B), lambda i: (0, i)),
        pl.BlockSpec((1, B), lambda i: (0, i)),
        pl.BlockSpec((NPAD, 2), lambda i: (0, 0)),
        pl.BlockSpec((NPAD, 1), lambda i: (0, 0)),
        _smem,
        _smem,
    ],
    out_specs=pl.BlockSpec((2, B), lambda i: (0, i)),
    out_shape=jax.ShapeDtypeStruct((2, NPAD), jnp.float32),
)


def _upd_body(co_ref, rep_ref, par_ref, alpha_ref, out_ref):
    fax = jnp.sum(par_ref[0], axis=0, keepdims=True)
    fay = jnp.sum(par_ref[1], axis=0, keepdims=True)
    a = alpha_ref[0, 0]
    dxt = a * (fax + rep_ref[0:1, :])
    dyt = a * (fay + rep_ref[1:2, :])
    nrm = jnp.sqrt(dxt * dxt + dyt * dyt + 1e-20)
    scale = jnp.minimum(CLAMP_STEP / (nrm + 1e-9), 1.0)
    out_ref[0:1, :] = co_ref[0:1, :] + dxt * scale
    out_ref[1:2, :] = co_ref[1:2, :] + dyt * scale


_upd = pl.pallas_call(
    _upd_body,
    in_specs=[
        pl.BlockSpec((2, NPAD), lambda: (0, 0)),
        pl.BlockSpec((2, NPAD), lambda: (0, 0)),
        pl.BlockSpec((2, NTILES, NPAD), lambda: (0, 0, 0)),
        _smem,
    ],
    out_specs=pl.BlockSpec((2, NPAD), lambda: (0, 0)),
    out_shape=jax.ShapeDtypeStruct((2, NPAD), jnp.float32),
)


def kernel(x, alpha, edge_index, batch):
    coT = jnp.pad(x[:, -2:], ((0, NPAD - N), (0, 0)))
    co = coT.T
    batch_p = jnp.pad(batch, (0, NPAD - N), constant_values=PAD_SENT)
    bf = batch_p.astype(jnp.float32)

    sqrtcnt, invcnt = _prep(batch_p.reshape(1, NPAD))

    tiles = bf.reshape(T, B)
    tmin = tiles[:, 0]
    tmax = tiles[:, -1]
    lo = jnp.searchsorted(tmax, tmin, side="left").astype(jnp.int32)
    hi = jnp.searchsorted(tmin, tmax, side="right").astype(jnp.int32)

    alpha_s = jnp.reshape(alpha, (1, 1)).astype(jnp.float32)
    sq1 = sqrtcnt.reshape(NPAD)
    bf2 = bf.reshape(1, NPAD)
    bfT = bf.reshape(NPAD, 1)

    for step in range(STEPS):
        par = _sc_attract_kernel()(co.reshape(2 * NPAD), sq1,
                                   edge_index.reshape(2 * E))
        rep = _rep(co, bf2, invcnt, coT, bfT, lo, hi)
        co = _upd(co, rep, par, alpha_s)
        if step + 1 < STEPS:
            coT = co.T
    return co[:, :N].T


# speedup vs baseline: 1.0954x; 1.0440x over previous
import functools

import jax
import jax.numpy as jnp
from jax import lax
from jax.experimental import pallas as pl
from jax.experimental.pallas import tpu as pltpu
from jax.experimental.pallas import tpu_sc as plsc

N = 10000
E = 320000
G = 100
STEPS = 3
EPS = 0.01
CLAMP_STEP = 0.1

B = 256
NPAD = 10240
T = NPAD // B
NTILES = 32
EPT = E // NTILES
LANES = 16
UNROLL = 5
PAD_SENT = 2 ** 30


def _prep_body(batch_ref, sqrtcnt_ref, invcnt_ref):
    b = batch_ref[...]
    g = lax.broadcasted_iota(jnp.int32, (G, 1), 0)
    m = (b == g).astype(jnp.float32)
    cnt_g = jnp.sum(m, axis=1, keepdims=True)
    cnode = jnp.sum(m * cnt_g, axis=0, keepdims=True)
    c = jnp.maximum(cnode, 1.0)
    sqrtcnt_ref[...] = jnp.sqrt(c)
    invcnt_ref[...] = 1.0 / c


_prep = pl.pallas_call(
    _prep_body,
    out_shape=(
        jax.ShapeDtypeStruct((1, NPAD), jnp.float32),
        jax.ShapeDtypeStruct((1, NPAD), jnp.float32),
    ),
)


def _sc_attract_body(co_hbm, sq_hbm, ei_hbm, out_hbm,
                     cov, sqv, rv, cv, fxv, fyv):
    wid = lax.axis_index("s") * 2 + lax.axis_index("c")
    base = wid * EPT
    pltpu.sync_copy(co_hbm, cov)
    pltpu.sync_copy(sq_hbm, sqv)
    pltpu.sync_copy(ei_hbm.at[pl.ds(base, EPT)], rv)
    pltpu.sync_copy(ei_hbm.at[pl.ds(E + base, EPT)], cv)

    zero16 = jnp.zeros((LANES,), jnp.float32)

    def _zero(i, carry):
        fxv[pl.ds(i * LANES, LANES)] = zero16
        fyv[pl.ds(i * LANES, LANES)] = zero16
        return carry

    lax.fori_loop(0, NPAD // LANES, _zero, 0)

    def _edges(i, carry):
        for u in range(UNROLL):
            o = (i * UNROLL + u) * LANES
            r = rv[pl.ds(o, LANES)]
            c = cv[pl.ds(o, LANES)]
            xr = plsc.load_gather(cov, [r])
            yr = plsc.load_gather(cov, [r + NPAD])
            xc = plsc.load_gather(cov, [c])
            yc = plsc.load_gather(cov, [c + NPAD])
            sq = plsc.load_gather(sqv, [r])
            dx = xr - xc
            dy = yr - yc
            d2 = dx * dx + dy * dy + 1e-20
            ib = plsc.bitcast(d2, jnp.int32)
            y = plsc.bitcast(jnp.int32(0x5F3759DF) - (ib >> 1), jnp.float32)
            h = 0.5 * d2
            y = y * (1.5 - h * y * y)
            y = y * (1.5 - h * y * y)
            coef = -(d2 * y + EPS) * sq
            ax = coef * dx
            ay = coef * dy
            plsc.addupdate_scatter(fxv, [r], ax)
            plsc.addupdate_scatter(fyv, [r], ay)
            plsc.addupdate_scatter(fxv, [c], -ax)
            plsc.addupdate_scatter(fyv, [c], -ay)
        return carry

    lax.fori_loop(0, EPT // (LANES * UNROLL), _edges, 0)

    pltpu.sync_copy(fxv, out_hbm.at[0, wid])
    pltpu.sync_copy(fyv, out_hbm.at[1, wid])


@functools.cache
def _sc_attract_kernel():
    mesh = plsc.VectorSubcoreMesh(core_axis_name="c", subcore_axis_name="s")
    return pl.kernel(
        _sc_attract_body,
        mesh=mesh,
        compiler_params=pltpu.CompilerParams(needs_layout_passes=False),
        out_type=jax.ShapeDtypeStruct((2, NTILES, NPAD), jnp.float32),
        scratch_types=[
            pltpu.VMEM((2 * NPAD,), jnp.float32),
            pltpu.VMEM((NPAD,), jnp.float32),
            pltpu.VMEM((EPT,), jnp.int32),
            pltpu.VMEM((EPT,), jnp.int32),
            pltpu.VMEM((NPAD,), jnp.float32),
            pltpu.VMEM((NPAD,), jnp.float32),
        ],
    )


def _rep_body(co_ref, bf_ref, ic_ref, coT_ref, bfT_ref, lo_ref, hi_ref,
              rep_ref):
    i = pl.program_id(0)
    xi = co_ref[0:1, :]
    yi = co_ref[1:2, :]
    bi = bf_ref[...]
    ki2 = ic_ref[...]

    def jbody(j, carry):
        sx, sy = carry
        off = j * B
        xj = coT_ref[pl.ds(off, B), 0:1]
        yj = coT_ref[pl.ds(off, B), 1:2]
        bj = bfT_ref[pl.ds(off, B), :]
        dx = xi - xj
        dy = yi - yj
        d2 = dx * dx + dy * dy
        dist = jnp.sqrt(d2) + EPS
        w = jnp.where(bi == bj, 1.0 / (dist * dist), 0.0)
        sx = sx + jnp.sum(w * dx, axis=0, keepdims=True)
        sy = sy + jnp.sum(w * dy, axis=0, keepdims=True)
        return sx, sy

    z = jnp.zeros((1, B), jnp.float32)
    sx, sy = lax.fori_loop(lo_ref[i], hi_ref[i], jbody, (z, z))
    rep_ref[0:1, :] = ki2 * sx
    rep_ref[1:2, :] = ki2 * sy


_smem = pl.BlockSpec(memory_space=pltpu.SMEM)

_rep = pl.pallas_call(
    _rep_body,
    grid=(T,),
    in_specs=[
        pl.BlockSpec((2, B), lambda i: (0, i)),
        pl.BlockSpec((1, B), lambda i: (0, i)),
        pl.BlockSpec((1, B), lambda i: (0, i)),
        pl.BlockSpec((NPAD, 2), lambda i: (0, 0)),
        pl.BlockSpec((NPAD, 1), lambda i: (0, 0)),
        _smem,
        _smem,
    ],
    out_specs=pl.BlockSpec((2, B), lambda i: (0, i)),
    out_shape=jax.ShapeDtypeStruct((2, NPAD), jnp.float32),
)


def _upd_body(co_ref, rep_ref, par_ref, alpha_ref, out_ref, outT_ref):
    fax = jnp.sum(par_ref[0], axis=0, keepdims=True)
    fay = jnp.sum(par_ref[1], axis=0, keepdims=True)
    a = alpha_ref[0, 0]
    dxt = a * (fax + rep_ref[0:1, :])
    dyt = a * (fay + rep_ref[1:2, :])
    nrm = jnp.sqrt(dxt * dxt + dyt * dyt + 1e-20)
    scale = jnp.minimum(CLAMP_STEP / (nrm + 1e-9), 1.0)
    nco = jnp.concatenate(
        [co_ref[0:1, :] + dxt * scale, co_ref[1:2, :] + dyt * scale], axis=0)
    out_ref[...] = nco
    outT_ref[...] = nco.T


_upd = pl.pallas_call(
    _upd_body,
    in_specs=[
        pl.BlockSpec((2, NPAD), lambda: (0, 0)),
        pl.BlockSpec((2, NPAD), lambda: (0, 0)),
        pl.BlockSpec((2, NTILES, NPAD), lambda: (0, 0, 0)),
        _smem,
    ],
    out_specs=(
        pl.BlockSpec((2, NPAD), lambda: (0, 0)),
        pl.BlockSpec((NPAD, 2), lambda: (0, 0)),
    ),
    out_shape=(
        jax.ShapeDtypeStruct((2, NPAD), jnp.float32),
        jax.ShapeDtypeStruct((NPAD, 2), jnp.float32),
    ),
)


def kernel(x, alpha, edge_index, batch):
    coT = jnp.pad(x[:, -2:], ((0, NPAD - N), (0, 0)))
    co = coT.T
    batch_p = jnp.pad(batch, (0, NPAD - N), constant_values=PAD_SENT)
    bf = batch_p.astype(jnp.float32)

    sqrtcnt, invcnt = _prep(batch_p.reshape(1, NPAD))

    tiles = bf.reshape(T, B)
    tmin = tiles[:, 0]
    tmax = tiles[:, -1]
    lo = jnp.searchsorted(tmax, tmin, side="left").astype(jnp.int32)
    hi = jnp.searchsorted(tmin, tmax, side="right").astype(jnp.int32)

    alpha_s = jnp.reshape(alpha, (1, 1)).astype(jnp.float32)
    sq1 = sqrtcnt.reshape(NPAD)
    bf2 = bf.reshape(1, NPAD)
    bfT = bf.reshape(NPAD, 1)

    for step in range(STEPS):
        par = _sc_attract_kernel()(co.reshape(2 * NPAD), sq1,
                                   edge_index.reshape(2 * E))
        rep = _rep(co, bf2, invcnt, coT, bfT, lo, hi)
        co, coT = _upd(co, rep, par, alpha_s)
    return coT[:N, :]


# speedup vs baseline: 1.1450x; 1.0453x over previous
import functools

import jax
import jax.numpy as jnp
from jax import lax
from jax.experimental import pallas as pl
from jax.experimental.pallas import tpu as pltpu
from jax.experimental.pallas import tpu_sc as plsc

N = 10000
E = 320000
G = 100
STEPS = 3
EPS = 0.01
CLAMP_STEP = 0.1

B = 256
NPAD = 10240
T = NPAD // B
NTILES = 32
EPT = E // NTILES
LANES = 16
UNROLL = 5
PAD_SENT = 2 ** 30


def _prep_body(batch_ref, bt_ref, sqrtcnt_ref, invcnt_ref,
               bf_ref, bfT_ref, lo_ref, hi_ref):
    b = batch_ref[...]
    g = lax.broadcasted_iota(jnp.int32, (G, 1), 0)
    m = (b == g).astype(jnp.float32)
    cnt_g = jnp.sum(m, axis=1, keepdims=True)
    cnode = jnp.sum(m * cnt_g, axis=0, keepdims=True)
    c = jnp.maximum(cnode, 1.0)
    sqrtcnt_ref[...] = jnp.sqrt(c)
    invcnt_ref[...] = 1.0 / c
    bf = b.astype(jnp.float32)
    bf_ref[...] = bf
    bfT_ref[...] = bf.T
    bt = bt_ref[...]
    tmin_c = bt[:, 0:1]
    tmax_c = bt[:, B - 1:B]
    tmin_r = tmin_c.T
    tmax_r = tmax_c.T
    lo_ref[...] = jnp.sum((tmax_c < tmin_r).astype(jnp.int32),
                          axis=0, keepdims=True)
    hi_ref[...] = jnp.sum((tmin_c <= tmax_r).astype(jnp.int32),
                          axis=0, keepdims=True)


_prep = pl.pallas_call(
    _prep_body,
    out_shape=(
        jax.ShapeDtypeStruct((1, NPAD), jnp.float32),
        jax.ShapeDtypeStruct((1, NPAD), jnp.float32),
        jax.ShapeDtypeStruct((1, NPAD), jnp.float32),
        jax.ShapeDtypeStruct((NPAD, 1), jnp.float32),
        jax.ShapeDtypeStruct((1, T), jnp.int32),
        jax.ShapeDtypeStruct((1, T), jnp.int32),
    ),
)


def _sc_attract_body(co_hbm, sq_hbm, ei_hbm, out_hbm,
                     cov, sqv, rv, cv, fxv, fyv):
    wid = lax.axis_index("s") * 2 + lax.axis_index("c")
    base = wid * EPT
    pltpu.sync_copy(co_hbm, cov)
    pltpu.sync_copy(sq_hbm, sqv)
    pltpu.sync_copy(ei_hbm.at[pl.ds(base, EPT)], rv)
    pltpu.sync_copy(ei_hbm.at[pl.ds(E + base, EPT)], cv)

    zero16 = jnp.zeros((LANES,), jnp.float32)

    def _zero(i, carry):
        fxv[pl.ds(i * LANES, LANES)] = zero16
        fyv[pl.ds(i * LANES, LANES)] = zero16
        return carry

    lax.fori_loop(0, NPAD // LANES, _zero, 0)

    def _edges(i, carry):
        for u in range(UNROLL):
            o = (i * UNROLL + u) * LANES
            r = rv[pl.ds(o, LANES)]
            c = cv[pl.ds(o, LANES)]
            zero = jnp.zeros((LANES,), jnp.int32)
            one = jnp.ones((LANES,), jnp.int32)
            xr = plsc.load_gather(cov, [zero, r])
            yr = plsc.load_gather(cov, [one, r])
            xc = plsc.load_gather(cov, [zero, c])
            yc = plsc.load_gather(cov, [one, c])
            sq = plsc.load_gather(sqv, [r])
            dx = xr - xc
            dy = yr - yc
            d2 = dx * dx + dy * dy + 1e-20
            ib = plsc.bitcast(d2, jnp.int32)
            y = plsc.bitcast(jnp.int32(0x5F3759DF) - (ib >> 1), jnp.float32)
            h = 0.5 * d2
            y = y * (1.5 - h * y * y)
            y = y * (1.5 - h * y * y)
            coef = -(d2 * y + EPS) * sq
            ax = coef * dx
            ay = coef * dy
            plsc.addupdate_scatter(fxv, [r], ax)
            plsc.addupdate_scatter(fyv, [r], ay)
            plsc.addupdate_scatter(fxv, [c], -ax)
            plsc.addupdate_scatter(fyv, [c], -ay)
        return carry

    lax.fori_loop(0, EPT // (LANES * UNROLL), _edges, 0)

    pltpu.sync_copy(fxv, out_hbm.at[0, wid])
    pltpu.sync_copy(fyv, out_hbm.at[1, wid])


@functools.cache
def _sc_attract_kernel():
    mesh = plsc.VectorSubcoreMesh(core_axis_name="c", subcore_axis_name="s")
    return pl.kernel(
        _sc_attract_body,
        mesh=mesh,
        compiler_params=pltpu.CompilerParams(needs_layout_passes=False),
        out_type=jax.ShapeDtypeStruct((2, NTILES, NPAD), jnp.float32),
        scratch_types=[
            pltpu.VMEM((2, NPAD), jnp.float32),
            pltpu.VMEM((NPAD,), jnp.float32),
            pltpu.VMEM((EPT,), jnp.int32),
            pltpu.VMEM((EPT,), jnp.int32),
            pltpu.VMEM((NPAD,), jnp.float32),
            pltpu.VMEM((NPAD,), jnp.float32),
        ],
    )


def _rep_body(co_ref, bf_ref, ic_ref, coT_ref, bfT_ref, lo_ref, hi_ref,
              rep_ref):
    i = pl.program_id(0)
    xi = co_ref[0:1, :]
    yi = co_ref[1:2, :]
    bi = bf_ref[...]
    ki2 = ic_ref[...]

    def jbody(j, carry):
        sx, sy = carry
        off = j * B
        xj = coT_ref[pl.ds(off, B), 0:1]
        yj = coT_ref[pl.ds(off, B), 1:2]
        bj = bfT_ref[pl.ds(off, B), :]
        dx = xi - xj
        dy = yi - yj
        d2 = dx * dx + dy * dy
        dist = jnp.sqrt(d2) + EPS
        w = jnp.where(bi == bj, 1.0 / (dist * dist), 0.0)
        sx = sx + jnp.sum(w * dx, axis=0, keepdims=True)
        sy = sy + jnp.sum(w * dy, axis=0, keepdims=True)
        return sx, sy

    z = jnp.zeros((1, B), jnp.float32)
    sx, sy = lax.fori_loop(lo_ref[0, i], hi_ref[0, i], jbody, (z, z))
    rep_ref[0:1, :] = ki2 * sx
    rep_ref[1:2, :] = ki2 * sy


_smem = pl.BlockSpec(memory_space=pltpu.SMEM)

_rep = pl.pallas_call(
    _rep_body,
    grid=(T,),
    in_specs=[
        pl.BlockSpec((2, B), lambda i: (0, i)),
        pl.BlockSpec((1, B), lambda i: (0, i)),
        pl.BlockSpec((1, B), lambda i: (0, i)),
        pl.BlockSpec((NPAD, 2), lambda i: (0, 0)),
        pl.BlockSpec((NPAD, 1), lambda i: (0, 0)),
        _smem,
        _smem,
    ],
    out_specs=pl.BlockSpec((2, B), lambda i: (0, i)),
    out_shape=jax.ShapeDtypeStruct((2, NPAD), jnp.float32),
)


def _upd_body(co_ref, rep_ref, par_ref, alpha_ref, out_ref, outT_ref):
    fax = jnp.sum(par_ref[0], axis=0, keepdims=True)
    fay = jnp.sum(par_ref[1], axis=0, keepdims=True)
    a = alpha_ref[0, 0]
    dxt = a * (fax + rep_ref[0:1, :])
    dyt = a * (fay + rep_ref[1:2, :])
    nrm = jnp.sqrt(dxt * dxt + dyt * dyt + 1e-20)
    scale = jnp.minimum(CLAMP_STEP / (nrm + 1e-9), 1.0)
    nco = jnp.concatenate(
        [co_ref[0:1, :] + dxt * scale, co_ref[1:2, :] + dyt * scale], axis=0)
    out_ref[...] = nco
    outT_ref[...] = nco.T


_upd = pl.pallas_call(
    _upd_body,
    in_specs=[
        pl.BlockSpec((2, NPAD), lambda: (0, 0)),
        pl.BlockSpec((2, NPAD), lambda: (0, 0)),
        pl.BlockSpec((2, NTILES, NPAD), lambda: (0, 0, 0)),
        _smem,
    ],
    out_specs=(
        pl.BlockSpec((2, NPAD), lambda: (0, 0)),
        pl.BlockSpec((NPAD, 2), lambda: (0, 0)),
    ),
    out_shape=(
        jax.ShapeDtypeStruct((2, NPAD), jnp.float32),
        jax.ShapeDtypeStruct((NPAD, 2), jnp.float32),
    ),
)


def kernel(x, alpha, edge_index, batch):
    coT = jnp.pad(x[:, -2:], ((0, NPAD - N), (0, 0)))
    co = coT.T
    batch_p = jnp.pad(batch, (0, NPAD - N), constant_values=PAD_SENT)

    sqrtcnt, invcnt, bf2, bfT, lo, hi = _prep(
        batch_p.reshape(1, NPAD), batch_p.reshape(T, B))

    alpha_s = jnp.reshape(alpha, (1, 1)).astype(jnp.float32)
    sq1 = sqrtcnt.reshape(NPAD)
    ei_flat = edge_index.reshape(2 * E)

    for step in range(STEPS):
        par = _sc_attract_kernel()(co, sq1, ei_flat)
        rep = _rep(co, bf2, invcnt, coT, bfT, lo, hi)
        co, coT = _upd(co, rep, par, alpha_s)
    return coT[:N, :]


# speedup vs baseline: 1.1811x; 1.0315x over previous
import functools

import jax
import jax.numpy as jnp
from jax import lax
from jax.experimental import pallas as pl
from jax.experimental.pallas import tpu as pltpu
from jax.experimental.pallas import tpu_sc as plsc

N = 10000
E = 320000
G = 100
STEPS = 3
EPS = 0.01
CLAMP_STEP = 0.1

B = 256
NPAD = 10240
T = NPAD // B
NTILES = 32
EPT = E // NTILES
EPAD = EPT + 240
LANES = 16
UNROLL = 5
PAD_SENT = 2 ** 30


def _prep_body(batch_ref, bt_ref, sqrtcnt_ref, invcnt_ref,
               bf_ref, bfT_ref, lo_ref, hi_ref):
    b = batch_ref[...]
    g = lax.broadcasted_iota(jnp.int32, (G, 1), 0)
    m = (b == g).astype(jnp.float32)
    cnt_g = jnp.sum(m, axis=1, keepdims=True)
    cnode = jnp.sum(m * cnt_g, axis=0, keepdims=True)
    c = jnp.maximum(cnode, 1.0)
    sqrtcnt_ref[...] = jnp.sqrt(c)
    invcnt_ref[...] = 1.0 / c
    bf = b.astype(jnp.float32)
    bf_ref[...] = bf
    bfT_ref[...] = bf.T
    bt = bt_ref[...]
    tmin_c = bt[:, 0:1]
    tmax_c = bt[:, B - 1:B]
    tmin_r = tmin_c.T
    tmax_r = tmax_c.T
    lo_ref[...] = jnp.sum((tmax_c < tmin_r).astype(jnp.int32),
                          axis=0, keepdims=True)
    hi_ref[...] = jnp.sum((tmin_c <= tmax_r).astype(jnp.int32),
                          axis=0, keepdims=True)


_prep = pl.pallas_call(
    _prep_body,
    out_shape=(
        jax.ShapeDtypeStruct((1, NPAD), jnp.float32),
        jax.ShapeDtypeStruct((1, NPAD), jnp.float32),
        jax.ShapeDtypeStruct((1, NPAD), jnp.float32),
        jax.ShapeDtypeStruct((NPAD, 1), jnp.float32),
        jax.ShapeDtypeStruct((1, T), jnp.int32),
        jax.ShapeDtypeStruct((1, T), jnp.int32),
    ),
)


def _sc_attract_body(co_hbm, sq_hbm, ei_hbm, out_hbm,
                     cov, sqv, ev, fxv, fyv):
    wid = lax.axis_index("s") * 2 + lax.axis_index("c")
    base = wid * EPT
    base_al = jnp.minimum((base // 128) * 128, E - EPAD)
    off0 = base - base_al
    pltpu.sync_copy(co_hbm, cov)
    pltpu.sync_copy(sq_hbm, sqv)
    pltpu.sync_copy(ei_hbm.at[:, pl.ds(base_al, EPAD)], ev)

    zero16 = jnp.zeros((LANES,), jnp.float32)

    def _zero(i, carry):
        fxv[pl.ds(i * LANES, LANES)] = zero16
        fyv[pl.ds(i * LANES, LANES)] = zero16
        return carry

    lax.fori_loop(0, NPAD // LANES, _zero, 0)

    def _edges(i, carry):
        for u in range(UNROLL):
            o = off0 + (i * UNROLL + u) * LANES
            r = ev[0, pl.ds(o, LANES)]
            c = ev[1, pl.ds(o, LANES)]
            zero = jnp.zeros((LANES,), jnp.int32)
            one = jnp.ones((LANES,), jnp.int32)
            xr = plsc.load_gather(cov, [zero, r])
            yr = plsc.load_gather(cov, [one, r])
            xc = plsc.load_gather(cov, [zero, c])
            yc = plsc.load_gather(cov, [one, c])
            sq = plsc.load_gather(sqv, [r])
            dx = xr - xc
            dy = yr - yc
            d2 = dx * dx + dy * dy + 1e-20
            ib = plsc.bitcast(d2, jnp.int32)
            y = plsc.bitcast(jnp.int32(0x5F3759DF) - (ib >> 1), jnp.float32)
            h = 0.5 * d2
            y = y * (1.5 - h * y * y)
            y = y * (1.5 - h * y * y)
            coef = -(d2 * y + EPS) * sq
            ax = coef * dx
            ay = coef * dy
            plsc.addupdate_scatter(fxv, [r], ax)
            plsc.addupdate_scatter(fyv, [r], ay)
            plsc.addupdate_scatter(fxv, [c], -ax)
            plsc.addupdate_scatter(fyv, [c], -ay)
        return carry

    lax.fori_loop(0, EPT // (LANES * UNROLL), _edges, 0)

    pltpu.sync_copy(fxv, out_hbm.at[0, wid])
    pltpu.sync_copy(fyv, out_hbm.at[1, wid])


@functools.cache
def _sc_attract_kernel():
    mesh = plsc.VectorSubcoreMesh(core_axis_name="c", subcore_axis_name="s")
    return pl.kernel(
        _sc_attract_body,
        mesh=mesh,
        compiler_params=pltpu.CompilerParams(needs_layout_passes=False),
        out_type=jax.ShapeDtypeStruct((2, NTILES, NPAD), jnp.float32),
        scratch_types=[
            pltpu.VMEM((2, NPAD), jnp.float32),
            pltpu.VMEM((NPAD,), jnp.float32),
            pltpu.VMEM((2, EPAD), jnp.int32),
            pltpu.VMEM((NPAD,), jnp.float32),
            pltpu.VMEM((NPAD,), jnp.float32),
        ],
    )


def _rep_body(co_ref, bf_ref, ic_ref, coT_ref, bfT_ref, lo_ref, hi_ref,
              rep_ref):
    i = pl.program_id(0)
    xi = co_ref[0:1, :]
    yi = co_ref[1:2, :]
    bi = bf_ref[...]
    ki2 = ic_ref[...]

    def jbody(j, carry):
        sx, sy = carry
        off = j * B
        xj = coT_ref[pl.ds(off, B), 0:1]
        yj = coT_ref[pl.ds(off, B), 1:2]
        bj = bfT_ref[pl.ds(off, B), :]
        dx = xi - xj
        dy = yi - yj
        d2 = dx * dx + dy * dy
        dist = jnp.sqrt(d2) + EPS
        w = jnp.where(bi == bj, 1.0 / (dist * dist), 0.0)
        sx = sx + jnp.sum(w * dx, axis=0, keepdims=True)
        sy = sy + jnp.sum(w * dy, axis=0, keepdims=True)
        return sx, sy

    z = jnp.zeros((1, B), jnp.float32)
    sx, sy = lax.fori_loop(lo_ref[0, i], hi_ref[0, i], jbody, (z, z))
    rep_ref[0:1, :] = ki2 * sx
    rep_ref[1:2, :] = ki2 * sy


_smem = pl.BlockSpec(memory_space=pltpu.SMEM)

_rep = pl.pallas_call(
    _rep_body,
    grid=(T,),
    in_specs=[
        pl.BlockSpec((2, B), lambda i: (0, i)),
        pl.BlockSpec((1, B), lambda i: (0, i)),
        pl.BlockSpec((1, B), lambda i: (0, i)),
        pl.BlockSpec((NPAD, 2), lambda i: (0, 0)),
        pl.BlockSpec((NPAD, 1), lambda i: (0, 0)),
        _smem,
        _smem,
    ],
    out_specs=pl.BlockSpec((2, B), lambda i: (0, i)),
    out_shape=jax.ShapeDtypeStruct((2, NPAD), jnp.float32),
)


def _upd_body(co_ref, rep_ref, par_ref, alpha_ref, out_ref, outT_ref):
    fax = jnp.sum(par_ref[0], axis=0, keepdims=True)
    fay = jnp.sum(par_ref[1], axis=0, keepdims=True)
    a = alpha_ref[0, 0]
    dxt = a * (fax + rep_ref[0:1, :])
    dyt = a * (fay + rep_ref[1:2, :])
    nrm = jnp.sqrt(dxt * dxt + dyt * dyt + 1e-20)
    scale = jnp.minimum(CLAMP_STEP / (nrm + 1e-9), 1.0)
    nco = jnp.concatenate(
        [co_ref[0:1, :] + dxt * scale, co_ref[1:2, :] + dyt * scale], axis=0)
    out_ref[...] = nco
    outT_ref[...] = nco.T


_upd = pl.pallas_call(
    _upd_body,
    in_specs=[
        pl.BlockSpec((2, NPAD), lambda: (0, 0)),
        pl.BlockSpec((2, NPAD), lambda: (0, 0)),
        pl.BlockSpec((2, NTILES, NPAD), lambda: (0, 0, 0)),
        _smem,
    ],
    out_specs=(
        pl.BlockSpec((2, NPAD), lambda: (0, 0)),
        pl.BlockSpec((NPAD, 2), lambda: (0, 0)),
    ),
    out_shape=(
        jax.ShapeDtypeStruct((2, NPAD), jnp.float32),
        jax.ShapeDtypeStruct((NPAD, 2), jnp.float32),
    ),
)


def kernel(x, alpha, edge_index, batch):
    coT = jnp.pad(x[:, -2:], ((0, NPAD - N), (0, 0)))
    co = coT.T
    batch_p = jnp.pad(batch, (0, NPAD - N), constant_values=PAD_SENT)

    sqrtcnt, invcnt, bf2, bfT, lo, hi = _prep(
        batch_p.reshape(1, NPAD), batch_p.reshape(T, B))

    alpha_s = jnp.reshape(alpha, (1, 1)).astype(jnp.float32)
    sq1 = sqrtcnt.reshape(NPAD)

    for step in range(STEPS):
        par = _sc_attract_kernel()(co, sq1, edge_index)
        rep = _rep(co, bf2, invcnt, coT, bfT, lo, hi)
        co, coT = _upd(co, rep, par, alpha_s)
    return coT[:N, :]


# speedup vs baseline: 1.2178x; 1.0311x over previous
import functools

import jax
import jax.numpy as jnp
from jax import lax
from jax.experimental import pallas as pl
from jax.experimental.pallas import tpu as pltpu
from jax.experimental.pallas import tpu_sc as plsc

N = 10000
E = 320000
G = 100
STEPS = 3
EPS = 0.01
CLAMP_STEP = 0.1

B = 256
KSTATIC = 3
NPAD = 10240
T = NPAD // B
NTILES = 32
EPT = E // NTILES
EPAD = EPT + 240
LANES = 16
UNROLL = 5
PAD_SENT = 2 ** 30


def _prep_body(batch_ref, bt_ref, sqrtcnt_ref, invcnt_ref,
               bf_ref, bfT_ref, lo_ref, hi_ref):
    b = batch_ref[...]
    g = lax.broadcasted_iota(jnp.int32, (G, 1), 0)
    m = (b == g).astype(jnp.float32)
    cnt_g = jnp.sum(m, axis=1, keepdims=True)
    cnode = jnp.sum(m * cnt_g, axis=0, keepdims=True)
    c = jnp.maximum(cnode, 1.0)
    sqrtcnt_ref[...] = jnp.sqrt(c)
    invcnt_ref[...] = 1.0 / c
    bf = b.astype(jnp.float32)
    bf_ref[...] = bf
    bfT_ref[...] = bf.T
    bt = bt_ref[...]
    tmin_c = bt[:, 0:1]
    tmax_c = bt[:, B - 1:B]
    tmin_r = tmin_c.T
    tmax_r = tmax_c.T
    lo_ref[...] = jnp.sum((tmax_c < tmin_r).astype(jnp.int32),
                          axis=0, keepdims=True)
    hi_ref[...] = jnp.sum((tmin_c <= tmax_r).astype(jnp.int32),
                          axis=0, keepdims=True)


_prep = pl.pallas_call(
    _prep_body,
    out_shape=(
        jax.ShapeDtypeStruct((1, NPAD), jnp.float32),
        jax.ShapeDtypeStruct((1, NPAD), jnp.float32),
        jax.ShapeDtypeStruct((1, NPAD), jnp.float32),
        jax.ShapeDtypeStruct((NPAD, 1), jnp.float32),
        jax.ShapeDtypeStruct((1, T), jnp.int32),
        jax.ShapeDtypeStruct((1, T), jnp.int32),
    ),
)


def _sc_attract_body(co_hbm, sq_hbm, ei_hbm, out_hbm,
                     cov, sqv, ev, fxv, fyv):
    wid = lax.axis_index("s") * 2 + lax.axis_index("c")
    base = wid * EPT
    base_al = jnp.minimum((base // 128) * 128, E - EPAD)
    off0 = base - base_al
    pltpu.sync_copy(co_hbm, cov)
    pltpu.sync_copy(sq_hbm, sqv)
    pltpu.sync_copy(ei_hbm.at[:, pl.ds(base_al, EPAD)], ev)

    zero16 = jnp.zeros((LANES,), jnp.float32)

    def _zero(i, carry):
        fxv[pl.ds(i * LANES, LANES)] = zero16
        fyv[pl.ds(i * LANES, LANES)] = zero16
        return carry

    lax.fori_loop(0, NPAD // LANES, _zero, 0)

    def _edges(i, carry):
        for u in range(UNROLL):
            o = off0 + (i * UNROLL + u) * LANES
            r = ev[0, pl.ds(o, LANES)]
            c = ev[1, pl.ds(o, LANES)]
            zero = jnp.zeros((LANES,), jnp.int32)
            one = jnp.ones((LANES,), jnp.int32)
            xr = plsc.load_gather(cov, [zero, r])
            yr = plsc.load_gather(cov, [one, r])
            xc = plsc.load_gather(cov, [zero, c])
            yc = plsc.load_gather(cov, [one, c])
            sq = plsc.load_gather(sqv, [r])
            dx = xr - xc
            dy = yr - yc
            d2 = dx * dx + dy * dy + 1e-20
            ib = plsc.bitcast(d2, jnp.int32)
            y = plsc.bitcast(jnp.int32(0x5F3759DF) - (ib >> 1), jnp.float32)
            h = 0.5 * d2
            y = y * (1.5 - h * y * y)
            y = y * (1.5 - h * y * y)
            coef = -(d2 * y + EPS) * sq
            ax = coef * dx
            ay = coef * dy
            plsc.addupdate_scatter(fxv, [r], ax)
            plsc.addupdate_scatter(fyv, [r], ay)
            plsc.addupdate_scatter(fxv, [c], -ax)
            plsc.addupdate_scatter(fyv, [c], -ay)
        return carry

    lax.fori_loop(0, EPT // (LANES * UNROLL), _edges, 0)

    pltpu.sync_copy(fxv, out_hbm.at[0, wid])
    pltpu.sync_copy(fyv, out_hbm.at[1, wid])


@functools.cache
def _sc_attract_kernel():
    mesh = plsc.VectorSubcoreMesh(core_axis_name="c", subcore_axis_name="s")
    return pl.kernel(
        _sc_attract_body,
        mesh=mesh,
        compiler_params=pltpu.CompilerParams(needs_layout_passes=False),
        out_type=jax.ShapeDtypeStruct((2, NTILES, NPAD), jnp.float32),
        scratch_types=[
            pltpu.VMEM((2, NPAD), jnp.float32),
            pltpu.VMEM((NPAD,), jnp.float32),
            pltpu.VMEM((2, EPAD), jnp.int32),
            pltpu.VMEM((NPAD,), jnp.float32),
            pltpu.VMEM((NPAD,), jnp.float32),
        ],
    )


def _rep_body(co_ref, bf_ref, ic_ref, coT_ref, bfT_ref, lo_ref, hi_ref,
              rep_ref):
    i = pl.program_id(0)
    xi = co_ref[0:1, :]
    yi = co_ref[1:2, :]
    bi = bf_ref[...]
    ki2 = ic_ref[...]

    def tile_sum(off, valid, sx, sy):
        xj = coT_ref[pl.ds(off, B), 0:1]
        yj = coT_ref[pl.ds(off, B), 1:2]
        bj = bfT_ref[pl.ds(off, B), :]
        dx = xi - xj
        dy = yi - yj
        d2 = dx * dx + dy * dy
        dist = jnp.sqrt(d2) + EPS
        w = jnp.where(bi == bj, valid / (dist * dist), 0.0)
        sx = sx + jnp.sum(w * dx, axis=0, keepdims=True)
        sy = sy + jnp.sum(w * dy, axis=0, keepdims=True)
        return sx, sy

    lo = lo_ref[0, i]
    hi = hi_ref[0, i]
    z = jnp.zeros((1, B), jnp.float32)
    sx, sy = z, z
    for k in range(KSTATIC):
        j = lo + k
        jc = jnp.minimum(j, T - 1)
        valid = (j < hi).astype(jnp.float32)
        sx, sy = tile_sum(jc * B, valid, sx, sy)

    def jbody(j, carry):
        return tile_sum(j * B, jnp.float32(1.0), *carry)

    sx, sy = lax.fori_loop(jnp.minimum(lo + KSTATIC, hi), hi, jbody, (sx, sy))
    rep_ref[0:1, :] = ki2 * sx
    rep_ref[1:2, :] = ki2 * sy


_smem = pl.BlockSpec(memory_space=pltpu.SMEM)

_rep = pl.pallas_call(
    _rep_body,
    grid=(T,),
    in_specs=[
        pl.BlockSpec((2, B), lambda i: (0, i)),
        pl.BlockSpec((1, B), lambda i: (0, i)),
        pl.BlockSpec((1, B), lambda i: (0, i)),
        pl.BlockSpec((NPAD, 2), lambda i: (0, 0)),
        pl.BlockSpec((NPAD, 1), lambda i: (0, 0)),
        _smem,
        _smem,
    ],
    out_specs=pl.BlockSpec((2, B), lambda i: (0, i)),
    out_shape=jax.ShapeDtypeStruct((2, NPAD), jnp.float32),
)


def _upd_body(co_ref, rep_ref, par_ref, alpha_ref, out_ref, outT_ref):
    fax = jnp.sum(par_ref[0], axis=0, keepdims=True)
    fay = jnp.sum(par_ref[1], axis=0, keepdims=True)
    a = alpha_ref[0, 0]
    dxt = a * (fax + rep_ref[0:1, :])
    dyt = a * (fay + rep_ref[1:2, :])
    nrm = jnp.sqrt(dxt * dxt + dyt * dyt + 1e-20)
    scale = jnp.minimum(CLAMP_STEP / (nrm + 1e-9), 1.0)
    nco = jnp.concatenate(
        [co_ref[0:1, :] + dxt * scale, co_ref[1:2, :] + dyt * scale], axis=0)
    out_ref[...] = nco
    outT_ref[...] = nco.T


_upd = pl.pallas_call(
    _upd_body,
    in_specs=[
        pl.BlockSpec((2, NPAD), lambda: (0, 0)),
        pl.BlockSpec((2, NPAD), lambda: (0, 0)),
        pl.BlockSpec((2, NTILES, NPAD), lambda: (0, 0, 0)),
        _smem,
    ],
    out_specs=(
        pl.BlockSpec((2, NPAD), lambda: (0, 0)),
        pl.BlockSpec((NPAD, 2), lambda: (0, 0)),
    ),
    out_shape=(
        jax.ShapeDtypeStruct((2, NPAD), jnp.float32),
        jax.ShapeDtypeStruct((NPAD, 2), jnp.float32),
    ),
)


def kernel(x, alpha, edge_index, batch):
    coT = jnp.pad(x[:, -2:], ((0, NPAD - N), (0, 0)))
    co = coT.T
    batch_p = jnp.pad(batch, (0, NPAD - N), constant_values=PAD_SENT)

    sqrtcnt, invcnt, bf2, bfT, lo, hi = _prep(
        batch_p.reshape(1, NPAD), batch_p.reshape(T, B))

    alpha_s = jnp.reshape(alpha, (1, 1)).astype(jnp.float32)
    sq1 = sqrtcnt.reshape(NPAD)

    for step in range(STEPS):
        par = _sc_attract_kernel()(co, sq1, edge_index)
        rep = _rep(co, bf2, invcnt, coT, bfT, lo, hi)
        co, coT = _upd(co, rep, par, alpha_s)
    return coT[:N, :]
